# Initial kernel scaffold; baseline (speedup 1.0000x reference)
import jax
import jax.numpy as jnp
from jax.experimental import pallas as pl


def kernel(x, edge_index, edge_attr, batch, emb_node, enc_W, enc_b, emb_edge, edge_W, edge_b, p_W1, p_b1, p_W2, p_b2, c_W1, c_b1, c_W2, c_b2, a_W1, a_b1, a_W2, a_b2, a_W3, a_b3, o_W1, o_b1, o_W2, o_b2):
    raise NotImplementedError("write your pallas kernel here")


# speedup vs baseline: 1.0805x; 1.0805x over previous
import functools

import jax
import jax.numpy as jnp
from jax.experimental import pallas as pl

D = 128
NUM_ITERS = 2
NUM_GRAPHS = 16

E_BLOCK = 1280
N_BLOCK = 2000


def _edge_mlp_kernel(attr_ref, xi_ref, xj_ref, wd_ref, ws_ref,
                     prep_ref, prec_ref, pw2_ref, cw2_ref, pb2_ref, cb2_ref,
                     mp_ref, mc_ref):
    xi = xi_ref[...]
    xj = xj_ref[...]
    u = jnp.dot(xi, wd_ref[...], preferred_element_type=jnp.float32)
    v = jnp.dot(xj, ws_ref[...], preferred_element_type=jnp.float32)
    a = attr_ref[0, 0, :][:, None]
    prep = prep_ref[...]
    prec = prec_ref[...]
    bp = jnp.where(a == 0, prep[0:1, :], jnp.where(a == 1, prep[1:2, :], prep[2:3, :]))
    bc = jnp.where(a == 0, prec[0:1, :], jnp.where(a == 1, prec[1:2, :], prec[2:3, :]))
    h1p = jax.nn.relu(u[:, :2 * D] + v[:, :2 * D] + bp)
    h1c = jax.nn.relu(u[:, 2 * D:] + v[:, 2 * D:] + bc)
    mp_ref[...] = jax.nn.relu(
        jnp.dot(h1p, pw2_ref[...], preferred_element_type=jnp.float32) + pb2_ref[0:1, :])
    mc_ref[...] = jax.nn.relu(
        jnp.dot(h1c, cw2_ref[...], preferred_element_type=jnp.float32) + cb2_ref[0:1, :])


def _edge_mlp(attr3d, xi, xj, wd, ws, prep, prec, pw2, cw2, pb2, cb2):
    e = xi.shape[0]
    nb = e // E_BLOCK
    grid = (nb,)
    return pl.pallas_call(
        _edge_mlp_kernel,
        grid=grid,
        in_specs=[
            pl.BlockSpec((1, 1, E_BLOCK), lambda i: (i, 0, 0)),
            pl.BlockSpec((E_BLOCK, D), lambda i: (i, 0)),
            pl.BlockSpec((E_BLOCK, D), lambda i: (i, 0)),
            pl.BlockSpec((D, 4 * D), lambda i: (0, 0)),
            pl.BlockSpec((D, 4 * D), lambda i: (0, 0)),
            pl.BlockSpec((8, 2 * D), lambda i: (0, 0)),
            pl.BlockSpec((8, 2 * D), lambda i: (0, 0)),
            pl.BlockSpec((2 * D, D), lambda i: (0, 0)),
            pl.BlockSpec((2 * D, D), lambda i: (0, 0)),
            pl.BlockSpec((8, D), lambda i: (0, 0)),
            pl.BlockSpec((8, D), lambda i: (0, 0)),
        ],
        out_specs=[
            pl.BlockSpec((E_BLOCK, D), lambda i: (i, 0)),
            pl.BlockSpec((E_BLOCK, D), lambda i: (i, 0)),
        ],
        out_shape=[
            jax.ShapeDtypeStruct((e, D), jnp.float32),
            jax.ShapeDtypeStruct((e, D), jnp.float32),
        ],
    )(attr3d, xi, xj, wd, ws, prep, prec, pw2, cw2, pb2, cb2)


def _node_update_kernel(nodes_ref, par_ref, chi_ref, w1n_ref, w1p_ref, w1c_ref,
                        b1_ref, w2_ref, b2_ref, w3_ref, b3_ref, out_ref):
    nodes = nodes_ref[...]
    h = jax.nn.relu(
        jnp.dot(nodes, w1n_ref[...], preferred_element_type=jnp.float32)
        + jnp.dot(par_ref[...], w1p_ref[...], preferred_element_type=jnp.float32)
        + jnp.dot(chi_ref[...], w1c_ref[...], preferred_element_type=jnp.float32)
        + b1_ref[0:1, :])
    h = jax.nn.relu(jnp.dot(h, w2_ref[...], preferred_element_type=jnp.float32) + b2_ref[0:1, :])
    h = jax.nn.relu(jnp.dot(h, w3_ref[...], preferred_element_type=jnp.float32) + b3_ref[0:1, :])
    out_ref[...] = nodes + h


def _node_update(nodes, par, chi, w1n, w1p, w1c, b1, w2, b2, w3, b3):
    n = nodes.shape[0]
    nb = n // N_BLOCK
    return pl.pallas_call(
        _node_update_kernel,
        grid=(nb,),
        in_specs=[
            pl.BlockSpec((N_BLOCK, D), lambda i: (i, 0)),
            pl.BlockSpec((N_BLOCK, D), lambda i: (i, 0)),
            pl.BlockSpec((N_BLOCK, D), lambda i: (i, 0)),
            pl.BlockSpec((D, 2 * D), lambda i: (0, 0)),
            pl.BlockSpec((D, 2 * D), lambda i: (0, 0)),
            pl.BlockSpec((D, 2 * D), lambda i: (0, 0)),
            pl.BlockSpec((8, 2 * D), lambda i: (0, 0)),
            pl.BlockSpec((2 * D, D), lambda i: (0, 0)),
            pl.BlockSpec((8, D), lambda i: (0, 0)),
            pl.BlockSpec((D, D), lambda i: (0, 0)),
            pl.BlockSpec((8, D), lambda i: (0, 0)),
        ],
        out_specs=pl.BlockSpec((N_BLOCK, D), lambda i: (i, 0)),
        out_shape=jax.ShapeDtypeStruct((n, D), jnp.float32),
    )(nodes, par, chi, w1n, w1p, w1c, b1, w2, b2, w3, b3)


def _output_kernel(batch_ref, x_ref, w1_ref, b1_ref, w2_ref, b2_ref, o_ref):
    i = pl.program_id(0)

    @pl.when(i == 0)
    def _():
        o_ref[...] = jnp.full((NUM_GRAPHS, 8 * D), -jnp.inf, jnp.float32)

    h = jax.nn.relu(jnp.dot(x_ref[...], w1_ref[...], preferred_element_type=jnp.float32)
                    + b1_ref[0:1, :])
    o = jax.nn.relu(jnp.dot(h, w2_ref[...], preferred_element_type=jnp.float32)
                    + b2_ref[0:1, :])
    b = batch_ref[0, 0, :][:, None]
    for g in range(NUM_GRAPHS):
        mg = jnp.max(jnp.where(b == g, o, -jnp.inf), axis=0)
        o_ref[g:g + 1, :] = jnp.maximum(o_ref[g:g + 1, :], mg[None, :])


def _output_mlp(batch3d, nodes, w1, b1, w2, b2):
    n = nodes.shape[0]
    nb = n // N_BLOCK
    return pl.pallas_call(
        _output_kernel,
        grid=(nb,),
        in_specs=[
            pl.BlockSpec((1, 1, N_BLOCK), lambda i: (i, 0, 0)),
            pl.BlockSpec((N_BLOCK, D), lambda i: (i, 0)),
            pl.BlockSpec((D, 4 * D), lambda i: (0, 0)),
            pl.BlockSpec((8, 4 * D), lambda i: (0, 0)),
            pl.BlockSpec((4 * D, 8 * D), lambda i: (0, 0)),
            pl.BlockSpec((8, 8 * D), lambda i: (0, 0)),
        ],
        out_specs=pl.BlockSpec((NUM_GRAPHS, 8 * D), lambda i: (0, 0)),
        out_shape=jax.ShapeDtypeStruct((NUM_GRAPHS, 8 * D), jnp.float32),
    )(batch3d, nodes, w1, b1, w2, b2)


def _pad8(x):
    r = x.reshape(1, -1) if x.ndim == 1 else x
    return jnp.pad(r, ((0, 8 - r.shape[0]), (0, 0)))


def kernel(x, edge_index, edge_attr, batch, emb_node, enc_W, enc_b, emb_edge,
           edge_W, edge_b, p_W1, p_b1, p_W2, p_b2, c_W1, c_b1, c_W2, c_b2,
           a_W1, a_b1, a_W2, a_b2, a_W3, a_b3, o_W1, o_b1, o_W2, o_b2):
    n = x.shape[0]
    e = edge_index.shape[1]

    node_table = emb_node @ enc_W
    table3 = jax.nn.relu(emb_edge @ edge_W + edge_b)
    prep = _pad8(table3 @ p_W1[2 * D:] + p_b1)
    prec = _pad8(table3 @ c_W1[2 * D:] + c_b1)
    wd = jnp.concatenate([p_W1[:D], c_W1[D:2 * D]], axis=1)
    ws = jnp.concatenate([p_W1[D:2 * D], c_W1[:D]], axis=1)

    nodes = jax.nn.relu(jnp.take(node_table, x, axis=0) + enc_b)

    src = edge_index[0]
    dst = edge_index[1]
    ones = jnp.ones((e,), jnp.float32)
    deg_dst = jax.ops.segment_sum(ones, dst, num_segments=n)
    deg_src = jax.ops.segment_sum(ones, src, num_segments=n)
    inv_dst = jnp.where(deg_dst > 0, 1.0 / deg_dst, 0.0)[:, None]
    inv_src = jnp.where(deg_src > 0, 1.0 / deg_src, 0.0)[:, None]

    attr3d = edge_attr.astype(jnp.int32).reshape(e // E_BLOCK, 1, E_BLOCK)
    batch3d = batch.astype(jnp.int32).reshape(n // N_BLOCK, 1, N_BLOCK)

    pb2 = _pad8(p_b2)
    cb2 = _pad8(c_b2)
    b1 = _pad8(a_b1)
    b2 = _pad8(a_b2)
    b3 = _pad8(a_b3)
    ob1 = _pad8(o_b1)
    ob2 = _pad8(o_b2)

    for _ in range(NUM_ITERS):
        xi = jnp.take(nodes, dst, axis=0)
        xj = jnp.take(nodes, src, axis=0)
        mp, mc = _edge_mlp(attr3d, xi, xj, wd, ws, prep, prec, p_W2, c_W2, pb2, cb2)
        par = inv_dst * jax.ops.segment_sum(mp, dst, num_segments=n)
        chi = inv_src * jax.ops.segment_sum(mc, src, num_segments=n)
        nodes = _node_update(nodes, par, chi,
                             a_W1[:D], a_W1[D:2 * D], a_W1[2 * D:],
                             b1, a_W2, b2, a_W3, b3)

    return _output_mlp(batch3d, nodes, o_W1, ob1, o_W2, ob2)


# speedup vs baseline: 3.0752x; 2.8461x over previous
import functools

import jax
import jax.numpy as jnp
from jax import lax
from jax.experimental import pallas as pl
from jax.experimental.pallas import tpu as pltpu
from jax.experimental.pallas import tpu_sc as plsc

D = 128
NUM_ITERS = 2
NUM_GRAPHS = 16

E_BLOCK = 1280
N_BLOCK = 2000

SC_CORES = 2
SC_SUBCORES = 16
CHUNK = 128


def _sc_mesh():
    return plsc.VectorSubcoreMesh(core_axis_name="c", subcore_axis_name="s")


def _sc_gather_pair(table, idx_a, idx_b):
    e = idx_a.shape[0]
    w = table.shape[1]
    nchunks = e // CHUNK
    per_sub = -(-nchunks // SC_SUBCORES)

    def body(table_hbm, ia_hbm, ib_hbm, oa_hbm, ob_hbm, idx_v, rows_v, sem):
        cid = lax.axis_index("c")
        sid = lax.axis_index("s")

        def sweep(i_hbm, o_hbm):
            @pl.loop(0, per_sub)
            def _(j):
                c = j * SC_SUBCORES + sid

                @pl.when(c < nchunks)
                def _():
                    base = c * CHUNK
                    pltpu.sync_copy(i_hbm.at[pl.ds(base, CHUNK)], idx_v)
                    pltpu.async_copy(table_hbm.at[idx_v], rows_v, sem).wait()
                    pltpu.sync_copy(rows_v, o_hbm.at[pl.ds(base, CHUNK)])

        @pl.when(cid == 0)
        def _():
            sweep(ia_hbm, oa_hbm)

        @pl.when(cid == 1)
        def _():
            sweep(ib_hbm, ob_hbm)

    f = pl.kernel(
        body,
        out_type=(jax.ShapeDtypeStruct((e, w), jnp.float32),
                  jax.ShapeDtypeStruct((e, w), jnp.float32)),
        mesh=_sc_mesh(),
        scratch_types=[
            pltpu.VMEM((CHUNK,), jnp.int32),
            pltpu.VMEM((CHUNK, w), jnp.float32),
            pltpu.SemaphoreType.DMA,
        ],
    )
    return f(table, idx_a, idx_b)


def _sc_scatter_add_pair(vals_a, vals_b, idx_a, idx_b, n_pad):
    e, w = vals_a.shape
    nchunks = e // CHUNK
    per_sub = -(-nchunks // SC_SUBCORES)
    rows_per_sub = n_pad // SC_SUBCORES

    def body(va_hbm, vb_hbm, ia_hbm, ib_hbm, oa_hbm, ob_hbm,
             idx_v, rows_v, acc_sh, sem):
        cid = lax.axis_index("c")
        sid = lax.axis_index("s")

        @pl.loop(0, CHUNK)
        def _(r):
            @pl.loop(0, w, step=16)
            def _(cc):
                rows_v[pl.ds(r, 1), pl.ds(cc, 16)] = jnp.zeros((1, 16), jnp.float32)

        base_row = sid * rows_per_sub

        @pl.loop(0, rows_per_sub, step=CHUNK)
        def _(r):
            pltpu.sync_copy(rows_v, acc_sh.at[pl.ds(base_row + r, CHUNK)])

        plsc.subcore_barrier()

        def sweep(v_hbm, i_hbm):
            @pl.loop(0, per_sub)
            def _(j):
                c = j * SC_SUBCORES + sid

                @pl.when(c < nchunks)
                def _():
                    base = c * CHUNK
                    pltpu.sync_copy(i_hbm.at[pl.ds(base, CHUNK)], idx_v)
                    pltpu.sync_copy(v_hbm.at[pl.ds(base, CHUNK)], rows_v)
                    pltpu.sync_copy(rows_v, acc_sh.at[idx_v], add=True)

        @pl.when(cid == 0)
        def _():
            sweep(va_hbm, ia_hbm)

        @pl.when(cid == 1)
        def _():
            sweep(vb_hbm, ib_hbm)

        plsc.subcore_barrier()

        @pl.when(cid == 0)
        def _():
            pltpu.sync_copy(acc_sh.at[pl.ds(base_row, rows_per_sub)],
                            oa_hbm.at[pl.ds(base_row, rows_per_sub)])

        @pl.when(cid == 1)
        def _():
            pltpu.sync_copy(acc_sh.at[pl.ds(base_row, rows_per_sub)],
                            ob_hbm.at[pl.ds(base_row, rows_per_sub)])

    f = pl.kernel(
        body,
        out_type=(jax.ShapeDtypeStruct((n_pad, w), jnp.float32),
                  jax.ShapeDtypeStruct((n_pad, w), jnp.float32)),
        mesh=_sc_mesh(),
        scratch_types=[
            pltpu.VMEM((CHUNK,), jnp.int32),
            pltpu.VMEM((CHUNK, w), jnp.float32),
            pltpu.VMEM_SHARED((n_pad, w), jnp.float32),
            pltpu.SemaphoreType.DMA,
        ],
    )
    return f(vals_a, vals_b, idx_a, idx_b)


def _edge_mlp_kernel(attr_ref, xi_ref, xj_ref, wd_ref, ws_ref,
                     prep_ref, prec_ref, pw2_ref, cw2_ref, pb2_ref, cb2_ref,
                     mp_ref, mc_ref):
    xi = xi_ref[...]
    xj = xj_ref[...]
    u = jnp.dot(xi, wd_ref[...], preferred_element_type=jnp.float32)
    v = jnp.dot(xj, ws_ref[...], preferred_element_type=jnp.float32)
    a = attr_ref[0, 0, :][:, None]
    prep = prep_ref[...]
    prec = prec_ref[...]
    bp = jnp.where(a == 0, prep[0:1, :], jnp.where(a == 1, prep[1:2, :], prep[2:3, :]))
    bc = jnp.where(a == 0, prec[0:1, :], jnp.where(a == 1, prec[1:2, :], prec[2:3, :]))
    h1p = jax.nn.relu(u[:, :2 * D] + v[:, :2 * D] + bp)
    h1c = jax.nn.relu(u[:, 2 * D:] + v[:, 2 * D:] + bc)
    mp_ref[...] = jax.nn.relu(
        jnp.dot(h1p, pw2_ref[...], preferred_element_type=jnp.float32) + pb2_ref[0:1, :])
    mc_ref[...] = jax.nn.relu(
        jnp.dot(h1c, cw2_ref[...], preferred_element_type=jnp.float32) + cb2_ref[0:1, :])


def _edge_mlp(attr3d, xi, xj, wd, ws, prep, prec, pw2, cw2, pb2, cb2):
    e = xi.shape[0]
    nb = e // E_BLOCK
    grid = (nb,)
    return pl.pallas_call(
        _edge_mlp_kernel,
        grid=grid,
        in_specs=[
            pl.BlockSpec((1, 1, E_BLOCK), lambda i: (i, 0, 0)),
            pl.BlockSpec((E_BLOCK, D), lambda i: (i, 0)),
            pl.BlockSpec((E_BLOCK, D), lambda i: (i, 0)),
            pl.BlockSpec((D, 4 * D), lambda i: (0, 0)),
            pl.BlockSpec((D, 4 * D), lambda i: (0, 0)),
            pl.BlockSpec((8, 2 * D), lambda i: (0, 0)),
            pl.BlockSpec((8, 2 * D), lambda i: (0, 0)),
            pl.BlockSpec((2 * D, D), lambda i: (0, 0)),
            pl.BlockSpec((2 * D, D), lambda i: (0, 0)),
            pl.BlockSpec((8, D), lambda i: (0, 0)),
            pl.BlockSpec((8, D), lambda i: (0, 0)),
        ],
        out_specs=[
            pl.BlockSpec((E_BLOCK, D), lambda i: (i, 0)),
            pl.BlockSpec((E_BLOCK, D), lambda i: (i, 0)),
        ],
        out_shape=[
            jax.ShapeDtypeStruct((e, D), jnp.float32),
            jax.ShapeDtypeStruct((e, D), jnp.float32),
        ],
    )(attr3d, xi, xj, wd, ws, prep, prec, pw2, cw2, pb2, cb2)


def _node_update_kernel(nodes_ref, par_ref, chi_ref, dd_ref, ds_ref,
                        w1n_ref, w1p_ref, w1c_ref,
                        b1_ref, w2_ref, b2_ref, w3_ref, b3_ref, out_ref):
    nodes = nodes_ref[...]
    dd = dd_ref[:, 0:1]
    ds = ds_ref[:, 0:1]
    par = jnp.where(dd > 0, 1.0 / dd, 0.0) * par_ref[...]
    chi = jnp.where(ds > 0, 1.0 / ds, 0.0) * chi_ref[...]
    h = jax.nn.relu(
        jnp.dot(nodes, w1n_ref[...], preferred_element_type=jnp.float32)
        + jnp.dot(par, w1p_ref[...], preferred_element_type=jnp.float32)
        + jnp.dot(chi, w1c_ref[...], preferred_element_type=jnp.float32)
        + b1_ref[0:1, :])
    h = jax.nn.relu(jnp.dot(h, w2_ref[...], preferred_element_type=jnp.float32) + b2_ref[0:1, :])
    h = jax.nn.relu(jnp.dot(h, w3_ref[...], preferred_element_type=jnp.float32) + b3_ref[0:1, :])
    out_ref[...] = nodes + h


def _node_update(nodes, par, chi, dd8, ds8, w1n, w1p, w1c, b1, w2, b2, w3, b3):
    n = nodes.shape[0]
    nb = n // N_BLOCK
    return pl.pallas_call(
        _node_update_kernel,
        grid=(nb,),
        in_specs=[
            pl.BlockSpec((N_BLOCK, D), lambda i: (i, 0)),
            pl.BlockSpec((N_BLOCK, D), lambda i: (i, 0)),
            pl.BlockSpec((N_BLOCK, D), lambda i: (i, 0)),
            pl.BlockSpec((N_BLOCK, 8), lambda i: (i, 0)),
            pl.BlockSpec((N_BLOCK, 8), lambda i: (i, 0)),
            pl.BlockSpec((D, 2 * D), lambda i: (0, 0)),
            pl.BlockSpec((D, 2 * D), lambda i: (0, 0)),
            pl.BlockSpec((D, 2 * D), lambda i: (0, 0)),
            pl.BlockSpec((8, 2 * D), lambda i: (0, 0)),
            pl.BlockSpec((2 * D, D), lambda i: (0, 0)),
            pl.BlockSpec((8, D), lambda i: (0, 0)),
            pl.BlockSpec((D, D), lambda i: (0, 0)),
            pl.BlockSpec((8, D), lambda i: (0, 0)),
        ],
        out_specs=pl.BlockSpec((N_BLOCK, D), lambda i: (i, 0)),
        out_shape=jax.ShapeDtypeStruct((n, D), jnp.float32),
    )(nodes, par, chi, dd8, ds8, w1n, w1p, w1c, b1, w2, b2, w3, b3)


def _output_kernel(batch_ref, x_ref, w1_ref, b1_ref, w2_ref, b2_ref, o_ref):
    i = pl.program_id(0)

    @pl.when(i == 0)
    def _():
        o_ref[...] = jnp.full((NUM_GRAPHS, 8 * D), -jnp.inf, jnp.float32)

    h = jax.nn.relu(jnp.dot(x_ref[...], w1_ref[...], preferred_element_type=jnp.float32)
                    + b1_ref[0:1, :])
    o = jax.nn.relu(jnp.dot(h, w2_ref[...], preferred_element_type=jnp.float32)
                    + b2_ref[0:1, :])
    b = batch_ref[0, 0, :][:, None]
    for g in range(NUM_GRAPHS):
        mg = jnp.max(jnp.where(b == g, o, -jnp.inf), axis=0)
        o_ref[g:g + 1, :] = jnp.maximum(o_ref[g:g + 1, :], mg[None, :])


def _output_mlp(batch3d, nodes, w1, b1, w2, b2):
    n = nodes.shape[0]
    nb = n // N_BLOCK
    return pl.pallas_call(
        _output_kernel,
        grid=(nb,),
        in_specs=[
            pl.BlockSpec((1, 1, N_BLOCK), lambda i: (i, 0, 0)),
            pl.BlockSpec((N_BLOCK, D), lambda i: (i, 0)),
            pl.BlockSpec((D, 4 * D), lambda i: (0, 0)),
            pl.BlockSpec((8, 4 * D), lambda i: (0, 0)),
            pl.BlockSpec((4 * D, 8 * D), lambda i: (0, 0)),
            pl.BlockSpec((8, 8 * D), lambda i: (0, 0)),
        ],
        out_specs=pl.BlockSpec((NUM_GRAPHS, 8 * D), lambda i: (0, 0)),
        out_shape=jax.ShapeDtypeStruct((NUM_GRAPHS, 8 * D), jnp.float32),
    )(batch3d, nodes, w1, b1, w2, b2)


def _pad8(x):
    r = x.reshape(1, -1) if x.ndim == 1 else x
    return jnp.pad(r, ((0, 8 - r.shape[0]), (0, 0)))


def kernel(x, edge_index, edge_attr, batch, emb_node, enc_W, enc_b, emb_edge,
           edge_W, edge_b, p_W1, p_b1, p_W2, p_b2, c_W1, c_b1, c_W2, c_b2,
           a_W1, a_b1, a_W2, a_b2, a_W3, a_b3, o_W1, o_b1, o_W2, o_b2):
    n = x.shape[0]
    e = edge_index.shape[1]

    node_table = emb_node @ enc_W
    table3 = jax.nn.relu(emb_edge @ edge_W + edge_b)
    prep = _pad8(table3 @ p_W1[2 * D:] + p_b1)
    prec = _pad8(table3 @ c_W1[2 * D:] + c_b1)
    wd = jnp.concatenate([p_W1[:D], c_W1[D:2 * D]], axis=1)
    ws = jnp.concatenate([p_W1[D:2 * D], c_W1[:D]], axis=1)

    n_pad = 10240
    x_pad = jnp.concatenate([x.astype(jnp.int32),
                             jnp.zeros((n_pad - n,), jnp.int32)])
    ng, _ = _sc_gather_pair(node_table, x_pad, x_pad)
    nodes = jax.nn.relu(ng[:n] + enc_b)

    src = edge_index[0].astype(jnp.int32)
    dst = edge_index[1].astype(jnp.int32)
    ones128 = jnp.ones((e, D), jnp.float32)
    dd, ds = _sc_scatter_add_pair(ones128, ones128, dst, src, n_pad)
    dd8 = dd[:n, :8]
    ds8 = ds[:n, :8]

    attr3d = edge_attr.astype(jnp.int32).reshape(e // E_BLOCK, 1, E_BLOCK)
    batch3d = batch.astype(jnp.int32).reshape(n // N_BLOCK, 1, N_BLOCK)

    pb2 = _pad8(p_b2)
    cb2 = _pad8(c_b2)
    b1 = _pad8(a_b1)
    b2 = _pad8(a_b2)
    b3 = _pad8(a_b3)
    ob1 = _pad8(o_b1)
    ob2 = _pad8(o_b2)

    for _ in range(NUM_ITERS):
        xi, xj = _sc_gather_pair(nodes, dst, src)
        mp, mc = _edge_mlp(attr3d, xi, xj, wd, ws, prep, prec, p_W2, c_W2, pb2, cb2)
        par, chi = _sc_scatter_add_pair(mp, mc, dst, src, n_pad)
        nodes = _node_update(nodes, par[:n], chi[:n], dd8, ds8,
                             a_W1[:D], a_W1[D:2 * D], a_W1[2 * D:],
                             b1, a_W2, b2, a_W3, b3)

    return _output_mlp(batch3d, nodes, o_W1, ob1, o_W2, ob2)
